# baseline (speedup 1.0000x reference)
import functools

import jax
import jax.numpy as jnp
from jax import lax
from jax.experimental import pallas as pl
from jax.experimental.pallas import tpu as pltpu
from jax.experimental.pallas import tpu_sc as plsc

_LANES = 16
_CHUNK = 128


def _make_kernel(B, D):
    info = plsc.get_sparse_core_info()
    nc, ns = info.num_cores, info.num_subcores
    nw = nc * ns
    bpw = B // nw
    nch = bpw // _CHUNK
    nd = D // _LANES

    mesh = plsc.VectorSubcoreMesh(core_axis_name="c", subcore_axis_name="s")

    @functools.partial(
        pl.kernel,
        mesh=mesh,
        compiler_params=pltpu.CompilerParams(needs_layout_passes=False),
        out_type=jax.ShapeDtypeStruct((B,), jnp.float32),
        scratch_types=[
            pltpu.VMEM((bpw,), jnp.int32),
            pltpu.VMEM((bpw,), jnp.int32),
            pltpu.VMEM((bpw,), jnp.int32),
            pltpu.VMEM((bpw,), jnp.int32),
            pltpu.VMEM((_CHUNK, 2 * D), jnp.float32),
            pltpu.VMEM((_CHUNK, 2 * D), jnp.float32),
            pltpu.VMEM((bpw,), jnp.float32),
            pltpu.SemaphoreType.DMA,
            pltpu.SemaphoreType.DMA,
        ],
    )
    def run(users_hbm, items_hbm, uemb_hbm, iemb_hbm, out_hbm,
            uidx, iidx, usup, isup, urows, irows, outv, usem, isem):
        wid = lax.axis_index("s") * nc + lax.axis_index("c")
        base = wid * bpw

        pltpu.sync_copy(users_hbm.at[pl.ds(base, bpw)], uidx)
        pltpu.sync_copy(items_hbm.at[pl.ds(base, bpw)], iidx)

        def sup(g, carry):
            sl = pl.ds(g * _LANES, _LANES)
            usup[sl] = lax.shift_right_logical(uidx[sl], 1)
            isup[sl] = lax.shift_right_logical(iidx[sl], 1)
            return carry

        lax.fori_loop(0, bpw // _LANES, sup, 0)

        lanes = lax.iota(jnp.int32, _LANES)

        def chunk(j, carry):
            cb = j * _CHUNK
            csl = pl.ds(cb, _CHUNK)
            cu = pltpu.async_copy(uemb_hbm.at[usup.at[csl]], urows, usem)
            ci = pltpu.async_copy(iemb_hbm.at[isup.at[csl]], irows, isem)
            cu.wait()
            ci.wait()

            for g16 in range(_CHUNK // _LANES):
                gb = g16 * _LANES
                upar = lax.bitwise_and(uidx[pl.ds(cb + gb, _LANES)], 1)
                ipar = lax.bitwise_and(iidx[pl.ds(cb + gb, _LANES)], 1)
                tot = jnp.zeros((_LANES,), jnp.float32)
                for b16 in range(_LANES):
                    k = gb + b16
                    pu = upar[b16] != 0
                    pi = ipar[b16] != 0
                    s = jnp.zeros((_LANES,), jnp.float32)
                    for c in range(nd):
                        ulo = urows[k, pl.ds(c * _LANES, _LANES)]
                        uhi = urows[k, pl.ds(D + c * _LANES, _LANES)]
                        ilo = irows[k, pl.ds(c * _LANES, _LANES)]
                        ihi = irows[k, pl.ds(D + c * _LANES, _LANES)]
                        s = s + (jnp.where(pu, uhi, ulo)
                                 * jnp.where(pi, ihi, ilo))
                    tot = jnp.where(lanes == b16, jnp.sum(s), tot)
                outv[pl.ds(cb + gb, _LANES)] = tot
            return carry

        lax.fori_loop(0, nch, chunk, 0)

        pltpu.sync_copy(outv, out_hbm.at[pl.ds(base, bpw)])

    return run


def kernel(users, items, user_emb, item_emb):
    B = users.shape[0]
    D = user_emb.shape[1]
    users = users.astype(jnp.int32)
    items = items.astype(jnp.int32)
    ue2 = user_emb.reshape(user_emb.shape[0] // 2, 2 * D)
    ie2 = item_emb.reshape(item_emb.shape[0] // 2, 2 * D)
    run = _make_kernel(B, D)
    return run(users, items, ue2, ie2)


# speedup vs baseline: 1.5526x; 1.5526x over previous
import functools

import jax
import jax.numpy as jnp
from jax import lax
from jax.experimental import pallas as pl
from jax.experimental.pallas import tpu as pltpu
from jax.experimental.pallas import tpu_sc as plsc

_LANES = 16
_CHUNK = 16


def _make_kernel(B, D):
    info = plsc.get_sparse_core_info()
    nc, ns = info.num_cores, info.num_subcores
    nw = nc * ns
    bpw = B // nw
    nd = D // _LANES
    nch = bpw // _CHUNK

    mesh = plsc.VectorSubcoreMesh(core_axis_name="c", subcore_axis_name="s")

    @functools.partial(
        pl.kernel,
        mesh=mesh,
        compiler_params=pltpu.CompilerParams(needs_layout_passes=False),
        out_type=jax.ShapeDtypeStruct((B,), jnp.float32),
        scratch_types=[
            pltpu.VMEM((bpw,), jnp.int32),
            pltpu.VMEM((bpw,), jnp.int32),
            pltpu.VMEM((_CHUNK, D), jnp.float32),
            pltpu.VMEM((_CHUNK, D), jnp.float32),
            pltpu.VMEM((bpw,), jnp.float32),
            pltpu.SemaphoreType.DMA,
            pltpu.SemaphoreType.DMA,
        ],
    )
    def run(users_hbm, items_hbm, uemb_hbm, iemb_hbm, out_hbm,
            uidx, iidx, ubuf, ibuf, outv, usem, isem):
        wid = lax.axis_index("s") * nc + lax.axis_index("c")
        base = wid * bpw

        pltpu.sync_copy(users_hbm.at[pl.ds(base, bpw)], uidx)
        pltpu.sync_copy(items_hbm.at[pl.ds(base, bpw)], iidx)

        lanes = lax.iota(jnp.int32, _LANES)

        def chunk(j, carry):
            cb = j * _CHUNK
            copies = []
            for g16 in range(_CHUNK // _LANES):
                uv = uidx[pl.ds(cb + g16 * _LANES, _LANES)]
                iv = iidx[pl.ds(cb + g16 * _LANES, _LANES)]
                for k in range(_LANES):
                    b = g16 * _LANES + k
                    copies.append(pltpu.async_copy(
                        uemb_hbm.at[pl.ds(uv[k], 1)],
                        ubuf.at[pl.ds(b, 1)], usem))
                    copies.append(pltpu.async_copy(
                        iemb_hbm.at[pl.ds(iv[k], 1)],
                        ibuf.at[pl.ds(b, 1)], isem))
            for c in copies:
                c.wait()

            for g16 in range(_CHUNK // _LANES):
                gb = g16 * _LANES
                tot = jnp.zeros((_LANES,), jnp.float32)
                for b16 in range(_LANES):
                    k = gb + b16
                    s = ubuf[k, pl.ds(0, _LANES)] * ibuf[k, pl.ds(0, _LANES)]
                    for c in range(1, nd):
                        s = s + (ubuf[k, pl.ds(c * _LANES, _LANES)]
                                 * ibuf[k, pl.ds(c * _LANES, _LANES)])
                    tot = jnp.where(lanes == b16, jnp.sum(s), tot)
                outv[pl.ds(cb + gb, _LANES)] = tot
            return carry

        lax.fori_loop(0, nch, chunk, 0)

        pltpu.sync_copy(outv, out_hbm.at[pl.ds(base, bpw)])

    return run


def kernel(users, items, user_emb, item_emb):
    B = users.shape[0]
    D = user_emb.shape[1]
    users = users.astype(jnp.int32)
    items = items.astype(jnp.int32)
    run = _make_kernel(B, D)
    return run(users, items, user_emb, item_emb)
